# baseline (speedup 1.0000x reference)
import functools

import jax
import jax.numpy as jnp
import numpy as np
from jax import lax
from jax.experimental import pallas as pl
from jax.experimental.pallas import tpu as pltpu
from jax.experimental.pallas import tpu_sc as plsc

_R = 3
_ND = 2 * _R + 1
_D_BM = _ND * _ND
_ND_TOT = _D_BM + 1
_K = 16
_HB = 16
_RANK_SENTINEL = 1 << 29


def _spiral_perm_np(r):
    n = 2 * r + 1
    order = [(0, 0)]
    x = y = 0
    dx, dy = 1, 0
    step = 1
    while len(order) < n * n:
        for _ in range(2):
            for _ in range(step):
                x += dx
                y += dy
                if abs(x) <= r and abs(y) <= r:
                    order.append((y, x))
            dx, dy = -dy, dx
        step += 1
    flat = [(yy + r) * n + (xx + r) for (yy, xx) in order]
    return np.array(flat, dtype=np.int32)


def _luts(w):
    perm = _spiral_perm_np(_R)
    rank = np.full((_ND_TOT,), _RANK_SENTINEL, dtype=np.int32)
    rank[perm] = np.arange(_D_BM, dtype=np.int32)
    rank_w = np.tile(rank[:, None], (1, w))
    perm_pad = np.zeros((64,), dtype=np.int32)
    perm_pad[: _D_BM] = perm
    rng = np.arange(-_R, _R + 1)
    jj, ii = np.meshgrid(rng, rng, indexing="ij")
    vl = (-1.0 * np.stack([jj, ii], axis=-1).reshape(-1, 2)).astype(np.float16)
    vl_pad = np.zeros((64, 2), dtype=np.float16)
    vl_pad[: _D_BM] = vl
    veclut_i32 = np.ascontiguousarray(vl_pad).view(np.int32).reshape(64)
    return list(int(v) for v in perm), rank_w, perm_pad, veclut_i32


def _f16_bits_to_int(b32):
    exp = lax.shift_right_logical(b32, 10)
    mant = jnp.bitwise_and(b32, 0x3FF)
    sh = jnp.bitwise_and(25 - exp, 31)
    return lax.shift_right_logical(jnp.bitwise_or(mant, 0x400), sh)


def _make_tc_body(perm_list, w):
    def body(w1_ref, w2_ref, rank_ref, minkey_ref, cost49_ref, tmpl_ref):
        bits1 = w1_ref[...].astype(jnp.int32)
        bits1 = bits1.reshape(_HB, _ND_TOT, _K, w)
        bits2 = w2_ref[...].astype(jnp.int32)
        v1 = _f16_bits_to_int(bits1)
        v2 = _f16_bits_to_int(bits2)
        diff = jnp.abs(v1 - v2[:, None])
        cost = jnp.sum(diff, axis=2)
        keys = cost * 64 + rank_ref[...][None]
        minkey = jnp.min(keys, axis=1)
        cost49 = cost[:, _D_BM]
        minkey_ref[...] = minkey
        cost49_ref[...] = cost49
        rankv = jnp.bitwise_and(minkey, 63)
        costbm = lax.shift_right_logical(minkey, 6)
        idxbm = jnp.full_like(rankv, perm_list[0])
        for s in range(1, _D_BM):
            idxbm = jnp.where(rankv == s, perm_list[s], idxbm)
        minidx = jnp.where(cost49 < costbm, _D_BM, idxbm)
        acc = jnp.zeros((_HB, _K, w), jnp.int32)
        for d in range(_ND_TOT):
            acc = jnp.where((minidx == d)[:, None, :], bits1[:, d], acc)
        tmpl_ref[...] = acc
    return body


def _tc_stage(w1p, w2p, rank_w, perm_list, nbh, w):
    grid = (nbh // _HB,)
    return pl.pallas_call(
        _make_tc_body(perm_list, w),
        grid=grid,
        in_specs=[
            pl.BlockSpec((_HB * _ND_TOT, _K, w), lambda i: (i, 0, 0)),
            pl.BlockSpec((_HB, _K, w), lambda i: (i, 0, 0)),
            pl.BlockSpec((_ND_TOT, w), lambda i: (0, 0)),
        ],
        out_specs=[
            pl.BlockSpec((_HB, w), lambda i: (i, 0)),
            pl.BlockSpec((_HB, w), lambda i: (i, 0)),
            pl.BlockSpec((_HB, _K, w), lambda i: (i, 0, 0)),
        ],
        out_shape=[
            jax.ShapeDtypeStruct((nbh, w), jnp.int32),
            jax.ShapeDtypeStruct((nbh, w), jnp.int32),
            jax.ShapeDtypeStruct((nbh, _K, w), jnp.int32),
        ],
        compiler_params=pltpu.CompilerParams(
            dimension_semantics=("arbitrary",)),
    )(w1p, w2p, rank_w)


def _sc_stage(minkey, cost49, perm_lut, vec_lut, n):
    nw = 32
    ch = n // nw
    g = ch // 16
    mesh = plsc.VectorSubcoreMesh(core_axis_name="c", subcore_axis_name="s")

    @functools.partial(
        pl.kernel,
        out_type=(
            jax.ShapeDtypeStruct((n,), jnp.int32),
            jax.ShapeDtypeStruct((n,), jnp.int32),
            jax.ShapeDtypeStruct((n,), jnp.int32),
        ),
        mesh=mesh,
        scratch_types=[
            pltpu.VMEM((64,), jnp.int32),
            pltpu.VMEM((64,), jnp.int32),
            pltpu.VMEM((ch,), jnp.int32),
            pltpu.VMEM((ch,), jnp.int32),
            pltpu.VMEM((ch,), jnp.int32),
            pltpu.VMEM((ch,), jnp.int32),
            pltpu.VMEM((ch,), jnp.int32),
        ],
        compiler_params=pltpu.CompilerParams(needs_layout_passes=False,
                                             use_tc_tiling_on_sc=False),
    )
    def sc_kernel(minkey_hbm, cost49_hbm, perm_hbm, veclut_hbm,
                  vec_out, mask_out, cost_out,
                  perm_v, veclut_v, mk_v, c49_v, vecb, maskb, costb):
        wid = lax.axis_index("s") * 2 + lax.axis_index("c")
        base = wid * ch
        pltpu.sync_copy(perm_hbm, perm_v)
        pltpu.sync_copy(veclut_hbm, veclut_v)
        pltpu.sync_copy(minkey_hbm.at[pl.ds(base, ch)], mk_v)
        pltpu.sync_copy(cost49_hbm.at[pl.ds(base, ch)], c49_v)

        def body(j, carry):
            sl = pl.ds(j * 16, 16)
            ki = mk_v[sl]
            rankv = jnp.bitwise_and(ki, 63)
            costbm = lax.shift_right_logical(ki, 6)
            idxbm = plsc.load_gather(perm_v, [rankv])
            c49 = c49_v[sl]
            mv = (c49 < costbm).astype(jnp.int32)
            vecb[sl] = plsc.load_gather(veclut_v, [idxbm])
            maskb[sl] = mv
            costb[sl] = jnp.minimum(costbm, c49)
            return carry

        lax.fori_loop(0, g, body, 0)
        pltpu.sync_copy(vecb, vec_out.at[pl.ds(base, ch)])
        pltpu.sync_copy(maskb, mask_out.at[pl.ds(base, ch)])
        pltpu.sync_copy(costb, cost_out.at[pl.ds(base, ch)])

    return sc_kernel(minkey, cost49, perm_lut, vec_lut)


def kernel(w1, w2):
    b, h, w, nd, k = w1.shape
    n = b * h * w
    nbh = b * h
    perm_list, rank_w, perm_pad, veclut_i32 = _luts(w)

    w1i = lax.bitcast_convert_type(w1, jnp.int16)
    w1p = jnp.transpose(w1i, (0, 1, 3, 4, 2)).reshape(nbh * nd, k, w)
    w2i = lax.bitcast_convert_type(w2, jnp.int16)
    w2p = jnp.transpose(w2i, (0, 1, 3, 4, 2)).reshape(nbh, k, w)

    minkey2d, cost492d, tmpl = _tc_stage(
        w1p, w2p, jnp.asarray(rank_w), perm_list, nbh, w)

    vec_i32, mask_i32, mincost = _sc_stage(
        minkey2d.reshape(n), cost492d.reshape(n),
        jnp.asarray(perm_pad), jnp.asarray(veclut_i32), n)

    vector = lax.bitcast_convert_type(vec_i32, jnp.float16).reshape(b, h, w, 2)
    tmpl_f16 = lax.bitcast_convert_type(
        tmpl.astype(jnp.uint16), jnp.float16)
    min_templates = jnp.transpose(
        tmpl_f16.reshape(b, h, k, w), (0, 1, 3, 2)).reshape(b, h, w, 1, k)
    input_mv_mask = (mask_i32 > 0).reshape(b, h, w, 1)
    min_cost = mincost.reshape(b, h, w, 1)
    return (vector, min_templates, input_mv_mask, min_cost)


# speedup vs baseline: 1.2023x; 1.2023x over previous
import functools

import jax
import jax.numpy as jnp
import numpy as np
from jax import lax
from jax.experimental import pallas as pl
from jax.experimental.pallas import tpu as pltpu
from jax.experimental.pallas import tpu_sc as plsc

_R = 3
_ND = 2 * _R + 1
_D_BM = _ND * _ND
_ND_TOT = _D_BM + 1
_K = 16
_HB = 8
_RANK_SENTINEL = 1 << 29


def _spiral_perm_np(r):
    n = 2 * r + 1
    order = [(0, 0)]
    x = y = 0
    dx, dy = 1, 0
    step = 1
    while len(order) < n * n:
        for _ in range(2):
            for _ in range(step):
                x += dx
                y += dy
                if abs(x) <= r and abs(y) <= r:
                    order.append((y, x))
            dx, dy = -dy, dx
        step += 1
    flat = [(yy + r) * n + (xx + r) for (yy, xx) in order]
    return np.array(flat, dtype=np.int32)


def _luts(w):
    perm = _spiral_perm_np(_R)
    rank = np.full((_ND_TOT,), float(_RANK_SENTINEL), dtype=np.float32)
    rank[perm] = np.arange(_D_BM, dtype=np.float32)
    rank_w = np.tile(rank[:, None], (1, w))
    perm_pad = np.zeros((64,), dtype=np.int32)
    perm_pad[: _D_BM] = perm
    rng = np.arange(-_R, _R + 1)
    jj, ii = np.meshgrid(rng, rng, indexing="ij")
    vl = (-1.0 * np.stack([jj, ii], axis=-1).reshape(-1, 2)).astype(np.float16)
    vl_pad = np.zeros((64, 2), dtype=np.float16)
    vl_pad[: _D_BM] = vl
    veclut_i32 = np.ascontiguousarray(vl_pad).view(np.int32).reshape(64)
    return list(int(v) for v in perm), rank_w, perm_pad, veclut_i32


def _f16_bits_to_f32(b32):
    f = lax.bitcast_convert_type(lax.shift_left(b32, 13), jnp.float32)
    return f * jnp.float32(5.192296858534828e33)


def _make_tc_body(perm_list, w):
    def body(w1_ref, w2_ref, rank_ref, minkey_ref, cost49_ref, tmpl_ref):
        bits1 = w1_ref[...].reshape(_HB, _ND_TOT, _K, w)
        v1 = _f16_bits_to_f32(bits1.astype(jnp.int32))
        v2 = _f16_bits_to_f32(w2_ref[...].astype(jnp.int32))
        diff = jnp.abs(v1 - v2[:, None])
        cost = jnp.sum(diff, axis=2)
        keys = cost * 64.0 + rank_ref[...][None]
        minkey = jnp.min(keys, axis=1).astype(jnp.int32)
        cost49 = cost[:, _D_BM].astype(jnp.int32)
        minkey_ref[...] = minkey
        cost49_ref[...] = cost49
        rankv = jnp.bitwise_and(minkey, 63)
        costbm = lax.shift_right_logical(minkey, 6)
        idxbm = jnp.full_like(rankv, perm_list[0])
        for s in range(1, _D_BM):
            idxbm = jnp.where(rankv == s, perm_list[s], idxbm)
        minidx = jnp.where(cost49 < costbm, _D_BM, idxbm)
        acc = jnp.zeros((_HB, _K, w), jnp.int16)
        for d in range(_ND_TOT):
            acc = jnp.where((minidx == d)[:, None, :], bits1[:, d], acc)
        tmpl_ref[...] = acc
    return body


def _tc_stage(w1p, w2p, rank_w, perm_list, nbh, w):
    grid = (nbh // _HB,)
    return pl.pallas_call(
        _make_tc_body(perm_list, w),
        grid=grid,
        in_specs=[
            pl.BlockSpec((_HB * _ND_TOT, _K, w), lambda i: (i, 0, 0)),
            pl.BlockSpec((_HB, _K, w), lambda i: (i, 0, 0)),
            pl.BlockSpec((_ND_TOT, w), lambda i: (0, 0)),
        ],
        out_specs=[
            pl.BlockSpec((_HB, w), lambda i: (i, 0)),
            pl.BlockSpec((_HB, w), lambda i: (i, 0)),
            pl.BlockSpec((_HB, _K, w), lambda i: (i, 0, 0)),
        ],
        out_shape=[
            jax.ShapeDtypeStruct((nbh, w), jnp.int32),
            jax.ShapeDtypeStruct((nbh, w), jnp.int32),
            jax.ShapeDtypeStruct((nbh, _K, w), jnp.int16),
        ],
        compiler_params=pltpu.CompilerParams(
            dimension_semantics=("arbitrary",)),
    )(w1p, w2p, rank_w)


def _sc_stage(minkey, cost49, perm_lut, vec_lut, n):
    nw = 32
    ch = n // nw
    g = ch // 16
    mesh = plsc.VectorSubcoreMesh(core_axis_name="c", subcore_axis_name="s")

    @functools.partial(
        pl.kernel,
        out_type=(
            jax.ShapeDtypeStruct((n,), jnp.int32),
            jax.ShapeDtypeStruct((n,), jnp.int32),
            jax.ShapeDtypeStruct((n,), jnp.int32),
        ),
        mesh=mesh,
        scratch_types=[
            pltpu.VMEM((64,), jnp.int32),
            pltpu.VMEM((64,), jnp.int32),
            pltpu.VMEM((ch,), jnp.int32),
            pltpu.VMEM((ch,), jnp.int32),
            pltpu.VMEM((ch,), jnp.int32),
            pltpu.VMEM((ch,), jnp.int32),
            pltpu.VMEM((ch,), jnp.int32),
        ],
        compiler_params=pltpu.CompilerParams(needs_layout_passes=False,
                                             use_tc_tiling_on_sc=False),
    )
    def sc_kernel(minkey_hbm, cost49_hbm, perm_hbm, veclut_hbm,
                  vec_out, mask_out, cost_out,
                  perm_v, veclut_v, mk_v, c49_v, vecb, maskb, costb):
        wid = lax.axis_index("s") * 2 + lax.axis_index("c")
        base = wid * ch
        pltpu.sync_copy(perm_hbm, perm_v)
        pltpu.sync_copy(veclut_hbm, veclut_v)
        pltpu.sync_copy(minkey_hbm.at[pl.ds(base, ch)], mk_v)
        pltpu.sync_copy(cost49_hbm.at[pl.ds(base, ch)], c49_v)

        def body(j, carry):
            sl = pl.ds(j * 16, 16)
            ki = mk_v[sl]
            rankv = jnp.bitwise_and(ki, 63)
            costbm = lax.shift_right_logical(ki, 6)
            idxbm = plsc.load_gather(perm_v, [rankv])
            c49 = c49_v[sl]
            mv = (c49 < costbm).astype(jnp.int32)
            vecb[sl] = plsc.load_gather(veclut_v, [idxbm])
            maskb[sl] = mv
            costb[sl] = jnp.minimum(costbm, c49)
            return carry

        lax.fori_loop(0, g, body, 0)
        pltpu.sync_copy(vecb, vec_out.at[pl.ds(base, ch)])
        pltpu.sync_copy(maskb, mask_out.at[pl.ds(base, ch)])
        pltpu.sync_copy(costb, cost_out.at[pl.ds(base, ch)])

    return sc_kernel(minkey, cost49, perm_lut, vec_lut)


def kernel(w1, w2):
    b, h, w, nd, k = w1.shape
    n = b * h * w
    nbh = b * h
    perm_list, rank_w, perm_pad, veclut_i32 = _luts(w)

    w1i = lax.bitcast_convert_type(w1, jnp.int16)
    w1p = jnp.transpose(w1i, (0, 1, 3, 4, 2)).reshape(nbh * nd, k, w)
    w2i = lax.bitcast_convert_type(w2, jnp.int16)
    w2p = jnp.transpose(w2i, (0, 1, 3, 4, 2)).reshape(nbh, k, w)

    minkey2d, cost492d, tmpl = _tc_stage(
        w1p, w2p, jnp.asarray(rank_w), perm_list, nbh, w)

    vec_i32, mask_i32, mincost = _sc_stage(
        minkey2d.reshape(n), cost492d.reshape(n),
        jnp.asarray(perm_pad), jnp.asarray(veclut_i32), n)

    vector = lax.bitcast_convert_type(vec_i32, jnp.float16).reshape(b, h, w, 2)
    tmpl_f16 = lax.bitcast_convert_type(tmpl, jnp.float16)
    min_templates = jnp.transpose(
        tmpl_f16.reshape(b, h, k, w), (0, 1, 3, 2)).reshape(b, h, w, 1, k)
    input_mv_mask = (mask_i32 > 0).reshape(b, h, w, 1)
    min_cost = mincost.reshape(b, h, w, 1)
    return (vector, min_templates, input_mv_mask, min_cost)
